# baseline (speedup 1.0000x reference)
import functools

import jax
import jax.numpy as jnp
from jax import lax
from jax.experimental import pallas as pl
from jax.experimental.pallas import tpu as pltpu
from jax.experimental.pallas import tpu_sc as plsc

_MOVE_PROB = 0.12
_NC, _NS = 2, 16
_NW = _NC * _NS


def _make_sc_gather(BT, C):
    RW = BT // _NW
    CH = 64
    NCH = RW // CH

    mesh = plsc.VectorSubcoreMesh(core_axis_name="c", subcore_axis_name="s")

    @functools.partial(
        pl.kernel,
        mesh=mesh,
        out_type=jax.ShapeDtypeStruct((BT, C), jnp.float32),
        scratch_types=[
            pltpu.VMEM((RW,), jnp.int32),
            pltpu.VMEM((CH, C), jnp.float32),
            pltpu.SemaphoreType.DMA,
        ],
    )
    def k(x_hbm, idx_hbm, out_hbm, idx_v, rows_v, sem):
        wid = lax.axis_index("s") * _NC + lax.axis_index("c")
        base = wid * RW
        pltpu.sync_copy(idx_hbm.at[pl.ds(base, RW)], idx_v)

        def chunk_body(j, carry):
            cbase = base + j * CH
            pltpu.async_copy(
                x_hbm.at[idx_v.at[pl.ds(j * CH, CH)]], rows_v, sem).wait()
            pltpu.sync_copy(rows_v, out_hbm.at[pl.ds(cbase, CH)])
            return carry

        lax.fori_loop(0, NCH, chunk_body, 0)

    return k


def kernel(x, training):
    B, T, C = x.shape
    BT = B * T
    xf = x.reshape(BT, C)

    def jittered(xf_):
        logp = jnp.log(jnp.array(
            [_MOVE_PROB / 2.0, 1.0 - _MOVE_PROB, _MOVE_PROB / 2.0],
            dtype=jnp.float32))
        step = jax.random.categorical(
            jax.random.key(42), logp, shape=(BT,)).astype(jnp.int32) - 1
        idx = jnp.arange(BT, dtype=jnp.int32) + step
        idx = idx + 2 * (idx < 0).astype(jnp.int32)
        idx = idx - 2 * (idx >= BT).astype(jnp.int32)
        return _make_sc_gather(BT, C)(xf_, idx)

    out = lax.cond(training != 0, jittered, lambda xf_: xf_, xf)
    return out.reshape(B, T, C)


# speedup vs baseline: 1.0308x; 1.0308x over previous
import functools

import jax
import jax.numpy as jnp
from jax import lax
from jax.experimental import pallas as pl
from jax.experimental.pallas import tpu as pltpu
from jax.experimental.pallas import tpu_sc as plsc

_MOVE_PROB = 0.12
_NC, _NS = 2, 16
_NW = _NC * _NS


def _make_sc_gather(BT, C):
    RW = BT // _NW
    CH = 64
    NCH = RW // CH

    mesh = plsc.VectorSubcoreMesh(core_axis_name="c", subcore_axis_name="s")

    @functools.partial(
        pl.kernel,
        mesh=mesh,
        out_type=jax.ShapeDtypeStruct((BT, C), jnp.float32),
        scratch_types=[
            pltpu.VMEM((RW,), jnp.int32),
            pltpu.VMEM((CH, C), jnp.float32),
            pltpu.VMEM((CH, C), jnp.float32),
            pltpu.SemaphoreType.DMA,
            pltpu.SemaphoreType.DMA,
            pltpu.SemaphoreType.DMA,
            pltpu.SemaphoreType.DMA,
        ],
    )
    def k(x_hbm, idx_hbm, out_hbm, idx_v, r_a, r_b, g_a, g_b, s_a, s_b):
        wid = lax.axis_index("s") * _NC + lax.axis_index("c")
        base = wid * RW
        pltpu.sync_copy(idx_hbm.at[pl.ds(base, RW)], idx_v)

        def start_g(c, buf, sem):
            pltpu.async_copy(x_hbm.at[idx_v.at[pl.ds(c * CH, CH)]], buf, sem)

        def wait_g(c, buf, sem):
            pltpu.make_async_copy(
                x_hbm.at[idx_v.at[pl.ds(c * CH, CH)]], buf, sem).wait()

        def start_s(c, buf, sem):
            pltpu.async_copy(buf, out_hbm.at[pl.ds(base + c * CH, CH)], sem)

        def wait_s(c, buf, sem):
            pltpu.make_async_copy(
                buf, out_hbm.at[pl.ds(base + c * CH, CH)], sem).wait()

        def step(c, buf, gsem, ssem, nbuf, ngsem, nssem, first, last):
            wait_g(c, buf, gsem)
            start_s(c, buf, ssem)
            if not last:
                if not first:
                    wait_s(c - 1, nbuf, nssem)
                start_g(c + 1, nbuf, ngsem)

        start_g(0, r_a, g_a)
        step(0, r_a, g_a, s_a, r_b, g_b, s_b, True, False)

        def body(jj, carry):
            c1 = 2 * jj + 1
            step(c1, r_b, g_b, s_b, r_a, g_a, s_a, False, False)
            step(c1 + 1, r_a, g_a, s_a, r_b, g_b, s_b, False, False)
            return carry

        lax.fori_loop(0, (NCH - 2) // 2, body, 0)
        step(NCH - 1, r_b, g_b, s_b, r_a, g_a, s_a, False, True)
        wait_s(NCH - 2, r_a, s_a)
        wait_s(NCH - 1, r_b, s_b)

    return k


def kernel(x, training):
    B, T, C = x.shape
    BT = B * T
    xf = x.reshape(BT, C)

    def jittered(xf_):
        logp = jnp.log(jnp.array(
            [_MOVE_PROB / 2.0, 1.0 - _MOVE_PROB, _MOVE_PROB / 2.0],
            dtype=jnp.float32))
        step = jax.random.categorical(
            jax.random.key(42), logp, shape=(BT,)).astype(jnp.int32) - 1
        idx = jnp.arange(BT, dtype=jnp.int32) + step
        idx = idx + 2 * (idx < 0).astype(jnp.int32)
        idx = idx - 2 * (idx >= BT).astype(jnp.int32)
        return _make_sc_gather(BT, C)(xf_, idx)

    out = lax.cond(training != 0, jittered, lambda xf_: xf_, xf)
    return out.reshape(B, T, C)


# speedup vs baseline: 2.3510x; 2.2808x over previous
import functools

import jax
import jax.numpy as jnp
from jax import lax
from jax.experimental import pallas as pl
from jax.experimental.pallas import tpu as pltpu
from jax.experimental.pallas import tpu_sc as plsc

_MOVE_PROB = 0.12
_NC, _NS = 2, 16
_NW = _NC * _NS


def _make_sc_gather(BT, C):
    RW = BT // _NW
    CH = 64
    NCH = RW // CH

    mesh = plsc.VectorSubcoreMesh(core_axis_name="c", subcore_axis_name="s")

    @functools.partial(
        pl.kernel,
        mesh=mesh,
        out_type=jax.ShapeDtypeStruct((BT, C), jnp.float32),
        scratch_types=[
            pltpu.VMEM((RW,), jnp.int32),
            pltpu.VMEM((CH, C), jnp.float32),
            pltpu.VMEM((CH, C), jnp.float32),
            pltpu.SemaphoreType.DMA,
            pltpu.SemaphoreType.DMA,
            pltpu.SemaphoreType.DMA,
            pltpu.SemaphoreType.DMA,
        ],
    )
    def k(x_hbm, idx_hbm, out_hbm, idx_v, r_a, r_b, g_a, g_b, s_a, s_b):
        wid = lax.axis_index("s") * _NC + lax.axis_index("c")
        base = wid * RW
        pltpu.sync_copy(idx_hbm.at[pl.ds(base, RW)], idx_v)

        def start_g(c, buf, sem):
            pltpu.async_copy(x_hbm.at[idx_v.at[pl.ds(c * CH, CH)]], buf, sem)

        def wait_g(c, buf, sem):
            pltpu.make_async_copy(
                x_hbm.at[idx_v.at[pl.ds(c * CH, CH)]], buf, sem).wait()

        def start_s(c, buf, sem):
            pltpu.async_copy(buf, out_hbm.at[pl.ds(base + c * CH, CH)], sem)

        def wait_s(c, buf, sem):
            pltpu.make_async_copy(
                buf, out_hbm.at[pl.ds(base + c * CH, CH)], sem).wait()

        def step(c, buf, gsem, ssem, nbuf, ngsem, nssem, first, last):
            wait_g(c, buf, gsem)
            start_s(c, buf, ssem)
            if not last:
                if not first:
                    wait_s(c - 1, nbuf, nssem)
                start_g(c + 1, nbuf, ngsem)

        start_g(0, r_a, g_a)
        step(0, r_a, g_a, s_a, r_b, g_b, s_b, True, False)

        def body(jj, carry):
            c1 = 2 * jj + 1
            step(c1, r_b, g_b, s_b, r_a, g_a, s_a, False, False)
            step(c1 + 1, r_a, g_a, s_a, r_b, g_b, s_b, False, False)
            return carry

        lax.fori_loop(0, (NCH - 2) // 2, body, 0)
        step(NCH - 1, r_b, g_b, s_b, r_a, g_a, s_a, False, True)
        wait_s(NCH - 2, r_a, s_a)
        wait_s(NCH - 1, r_b, s_b)

    return k


def kernel(x, training):
    B, T, C = x.shape
    BT = B * T
    xf = x.reshape(BT, C)

    logp = jnp.log(jnp.array(
        [_MOVE_PROB / 2.0, 1.0 - _MOVE_PROB, _MOVE_PROB / 2.0],
        dtype=jnp.float32))
    step = jax.random.categorical(
        jax.random.key(42), logp, shape=(BT,)).astype(jnp.int32) - 1
    iota = jnp.arange(BT, dtype=jnp.int32)
    idx = iota + step
    idx = idx + 2 * (idx < 0).astype(jnp.int32)
    idx = idx - 2 * (idx >= BT).astype(jnp.int32)
    idx = jnp.where(training != 0, idx, iota)
    out = _make_sc_gather(BT, C)(xf, idx)
    return out.reshape(B, T, C)


# speedup vs baseline: 2.4136x; 1.0266x over previous
import functools

import jax
import jax.numpy as jnp
from jax import lax
from jax.experimental import pallas as pl
from jax.experimental.pallas import tpu as pltpu
from jax.experimental.pallas import tpu_sc as plsc

_MOVE_PROB = 0.12
_NC, _NS = 2, 16
_NW = _NC * _NS


def _make_sc_gather(BT, C):
    RW = BT // _NW
    CH = 32
    NBUF = 4
    NCH = RW // CH

    mesh = plsc.VectorSubcoreMesh(core_axis_name="c", subcore_axis_name="s")

    @functools.partial(
        pl.kernel,
        mesh=mesh,
        out_type=jax.ShapeDtypeStruct((BT, C), jnp.float32),
        scratch_types=(
            [pltpu.VMEM((RW,), jnp.int32)]
            + [pltpu.VMEM((CH, C), jnp.float32)] * NBUF
            + [pltpu.SemaphoreType.DMA] * NBUF
            + [pltpu.SemaphoreType.DMA] * NBUF
        ),
    )
    def k(x_hbm, idx_hbm, out_hbm, idx_v, *bufsem):
        bufs = bufsem[:NBUF]
        gsem = bufsem[NBUF:2 * NBUF]
        ssem = bufsem[2 * NBUF:]
        wid = lax.axis_index("s") * _NC + lax.axis_index("c")
        base = wid * RW
        pltpu.sync_copy(idx_hbm.at[pl.ds(base, RW)], idx_v)

        def start_g(c, b):
            pltpu.async_copy(
                x_hbm.at[idx_v.at[pl.ds(c * CH, CH)]], bufs[b], gsem[b])

        def wait_g(c, b):
            pltpu.make_async_copy(
                x_hbm.at[idx_v.at[pl.ds(c * CH, CH)]], bufs[b], gsem[b]).wait()

        def start_s(c, b):
            pltpu.async_copy(
                bufs[b], out_hbm.at[pl.ds(base + c * CH, CH)], ssem[b])

        def wait_s(c, b):
            pltpu.make_async_copy(
                bufs[b], out_hbm.at[pl.ds(base + c * CH, CH)], ssem[b]).wait()

        start_g(0, 0)
        start_g(1, 1)

        def step(c, b, first, last):
            wait_g(c, b)
            start_s(c, b)
            if not last:
                bn = (b + 2) % NBUF
                if not first:
                    wait_s(c - 2, bn)
                start_g(c + 2, bn)

        step(0, 0, True, False)
        step(1, 1, True, False)

        def body(jj, carry):
            c0 = 4 * jj + 2
            for b in range(NBUF):
                step(c0 + b, (2 + b) % NBUF, False, False)
            return carry

        lax.fori_loop(0, (NCH - 4) // 4, body, 0)
        step(NCH - 2, (NCH - 2) % NBUF, False, True)
        step(NCH - 1, (NCH - 1) % NBUF, False, True)
        for c in range(NCH - 4, NCH):
            wait_s(c, c % NBUF)

    return k


def kernel(x, training):
    B, T, C = x.shape
    BT = B * T
    xf = x.reshape(BT, C)

    logp = jnp.log(jnp.array(
        [_MOVE_PROB / 2.0, 1.0 - _MOVE_PROB, _MOVE_PROB / 2.0],
        dtype=jnp.float32))
    step = jax.random.categorical(
        jax.random.key(42), logp, shape=(BT,)).astype(jnp.int32) - 1
    iota = jnp.arange(BT, dtype=jnp.int32)
    idx = iota + step
    idx = idx + 2 * (idx < 0).astype(jnp.int32)
    idx = idx - 2 * (idx >= BT).astype(jnp.int32)
    idx = jnp.where(training != 0, idx, iota)
    out = _make_sc_gather(BT, C)(xf, idx)
    return out.reshape(B, T, C)


# speedup vs baseline: 2.4487x; 1.0145x over previous
import functools

import jax
import jax.numpy as jnp
from jax import lax
from jax.experimental import pallas as pl
from jax.experimental.pallas import tpu as pltpu
from jax.experimental.pallas import tpu_sc as plsc

_MOVE_PROB = 0.12
_NC, _NS = 2, 16
_NW = _NC * _NS


def _make_sc_gather(BT, C):
    RW = BT // _NW
    CH = 16
    NBUF = 8
    AHD = 4
    NCH = RW // CH

    mesh = plsc.VectorSubcoreMesh(core_axis_name="c", subcore_axis_name="s")

    @functools.partial(
        pl.kernel,
        mesh=mesh,
        out_type=jax.ShapeDtypeStruct((BT, C), jnp.float32),
        scratch_types=(
            [pltpu.VMEM((RW,), jnp.int32)]
            + [pltpu.VMEM((CH, C), jnp.float32)] * NBUF
            + [pltpu.SemaphoreType.DMA] * NBUF
            + [pltpu.SemaphoreType.DMA] * NBUF
        ),
    )
    def k(x_hbm, idx_hbm, out_hbm, idx_v, *bufsem):
        bufs = bufsem[:NBUF]
        gsem = bufsem[NBUF:2 * NBUF]
        ssem = bufsem[2 * NBUF:]
        wid = lax.axis_index("s") * _NC + lax.axis_index("c")
        base = wid * RW
        pltpu.sync_copy(idx_hbm.at[pl.ds(base, RW)], idx_v)

        def start_g(c, b):
            pltpu.async_copy(
                x_hbm.at[idx_v.at[pl.ds(c * CH, CH)]], bufs[b], gsem[b])

        def wait_g(c, b):
            pltpu.make_async_copy(
                x_hbm.at[idx_v.at[pl.ds(c * CH, CH)]], bufs[b], gsem[b]).wait()

        def start_s(c, b):
            pltpu.async_copy(
                bufs[b], out_hbm.at[pl.ds(base + c * CH, CH)], ssem[b])

        def wait_s(c, b):
            pltpu.make_async_copy(
                bufs[b], out_hbm.at[pl.ds(base + c * CH, CH)], ssem[b]).wait()

        for c in range(AHD):
            start_g(c, c)

        def step(c, b, first, last):
            wait_g(c, b)
            start_s(c, b)
            if not last:
                bn = (b + AHD) % NBUF
                if not first:
                    wait_s(c + AHD - NBUF, bn)
                start_g(c + AHD, bn)

        for c in range(NBUF - AHD):
            step(c, c, True, False)

        def body(jj, carry):
            c0 = NBUF * jj + (NBUF - AHD)
            for b in range(NBUF):
                step(c0 + b, (NBUF - AHD + b) % NBUF, False, False)
            return carry

        lax.fori_loop(0, (NCH - NBUF) // NBUF, body, 0)
        for c in range(NCH - AHD, NCH):
            step(c, c % NBUF, False, True)
        for c in range(NCH - NBUF, NCH):
            wait_s(c, c % NBUF)

    return k


def kernel(x, training):
    B, T, C = x.shape
    BT = B * T
    xf = x.reshape(BT, C)

    logp = jnp.log(jnp.array(
        [_MOVE_PROB / 2.0, 1.0 - _MOVE_PROB, _MOVE_PROB / 2.0],
        dtype=jnp.float32))
    step = jax.random.categorical(
        jax.random.key(42), logp, shape=(BT,)).astype(jnp.int32) - 1
    iota = jnp.arange(BT, dtype=jnp.int32)
    idx = iota + step
    idx = idx + 2 * (idx < 0).astype(jnp.int32)
    idx = idx - 2 * (idx >= BT).astype(jnp.int32)
    idx = jnp.where(training != 0, idx, iota)
    out = _make_sc_gather(BT, C)(xf, idx)
    return out.reshape(B, T, C)
